# baseline (speedup 1.0000x reference)
import functools

import jax
import jax.numpy as jnp
from jax import lax
from jax.experimental import pallas as pl
from jax.experimental.pallas import tpu as pltpu
from jax.experimental.pallas import tpu_sc as plsc

EMB_DIM = 32
SCALE = float(EMB_DIM ** 0.5)

NUM_CORES = 2
NUM_SUBCORES = 16
NUM_WORKERS = NUM_CORES * NUM_SUBCORES
LANES = 16

IDX_PER_STREAM = 128
STREAMS_PER_UNIT = 4
UNIT = IDX_PER_STREAM * STREAMS_PER_UNIT
S8_PER_UNIT = 4
TILE_WORDS = 8 * 128
UNIT_TILES = S8_PER_UNIT * (EMB_DIM // 8)


@functools.partial(jax.jit, static_argnames=("b", "s"))
def _embed_native(idx_flat, weight, *, b, s):
    total = b * s
    n_per_w = total // NUM_WORKERS
    n_units = n_per_w // UNIT
    n_pairs = n_units // 2
    assert n_pairs * 2 * UNIT == n_per_w
    b_tiles = b // 128
    d_tiles = EMB_DIM // 8
    units_per_tile = 8 // S8_PER_UNIT
    out_rows = s * d_tiles * b_tiles
    mesh = plsc.VectorSubcoreMesh(core_axis_name="c", subcore_axis_name="s")

    @functools.partial(
        pl.kernel,
        mesh=mesh,
        compiler_params=pltpu.CompilerParams(
            use_tc_tiling_on_sc=False, needs_layout_passes=False),
        out_type=jax.ShapeDtypeStruct((out_rows, 8, 128), jnp.float32),
        scratch_types=[
            pltpu.VMEM((UNIT,), jnp.int32),
            pltpu.VMEM((UNIT,), jnp.int32),
            pltpu.VMEM((UNIT, EMB_DIM), jnp.float32),
            pltpu.VMEM((UNIT, EMB_DIM), jnp.float32),
            pltpu.VMEM((UNIT_TILES * 8, 129), jnp.float32),
            pltpu.VMEM((UNIT_TILES * 8, 129), jnp.float32),
            pltpu.SemaphoreType.DMA,
            pltpu.SemaphoreType.DMA,
            pltpu.SemaphoreType.DMA,
            pltpu.SemaphoreType.DMA,
        ],
    )
    def k(idx_hbm, table_hbm, out_hbm, idx_a, idx_b, rows_a, rows_b,
          tile_a, tile_b, gsem_a, gsem_b, wsem_a, wsem_b):
        wid = lax.axis_index("s") * NUM_CORES + lax.axis_index("c")
        unit0 = wid * n_units

        def fire_gathers(idx_v, rows_v, sem):
            for t in range(STREAMS_PER_UNIT):
                sl = pl.ds(t * IDX_PER_STREAM, IDX_PER_STREAM)
                pltpu.async_copy(table_hbm.at[idx_v.at[sl]], rows_v.at[sl], sem)

        def drain_gathers(idx_v, rows_v, sem):
            for t in range(STREAMS_PER_UNIT):
                sl = pl.ds(t * IDX_PER_STREAM, IDX_PER_STREAM)
                pltpu.make_async_copy(
                    table_hbm.at[idx_v.at[sl]], rows_v.at[sl], sem).wait()

        lane = lax.iota(jnp.int32, LANES)

        def transpose_scale(rows_v, tile_v):
            @plsc.parallel_loop(0, IDX_PER_STREAM, 1, unroll=4,
                                carry=jnp.zeros((LANES,), jnp.int32))
            def body(t128, col):
                for s8 in range(S8_PER_UNIT):
                    tok = s8 * IDX_PER_STREAM + t128
                    for h in range(2):
                        v = rows_v[tok, pl.ds(h * LANES, LANES)]
                        plsc.store_scatter(
                            tile_v, [lane + (s8 * 32 + h * LANES), col], v)
                return col + 1

        def unit_coords(u):
            ts = u // (b_tiles * units_per_tile)
            r = lax.rem(u, b_tiles * units_per_tile)
            tb = r // units_per_tile
            h = lax.rem(r, units_per_tile)
            return ts, tb, h

        def unit_writes(tile_v, u, sem, fire):
            ts, tb, h = unit_coords(u)
            for s8 in range(S8_PER_UNIT):
                s_row = ts * 8 + h * S8_PER_UNIT + s8
                for td in range(d_tiles):
                    src = tile_v.at[pl.ds((s8 * d_tiles + td) * 8, 8),
                                    pl.ds(0, 128)]
                    dst = out_hbm.at[(s_row * d_tiles + td) * b_tiles + tb]
                    if fire:
                        pltpu.async_copy(src, dst, sem)
                    else:
                        pltpu.make_async_copy(src, dst, sem).wait()

        pltpu.sync_copy(idx_hbm.at[pl.ds(unit0 * UNIT, UNIT)], idx_a)
        fire_gathers(idx_a, rows_a, gsem_a)

        def pair_body(j, carry):
            ua = unit0 + 2 * j
            ub = ua + 1

            @pl.when(j > 0)
            def _():
                unit_writes(tile_b, ub - 2, wsem_b, fire=False)

            pltpu.sync_copy(idx_hbm.at[pl.ds(ub * UNIT, UNIT)], idx_b)
            fire_gathers(idx_b, rows_b, gsem_b)

            @pl.when(j > 0)
            def _():
                unit_writes(tile_a, ua - 2, wsem_a, fire=False)

            drain_gathers(idx_a, rows_a, gsem_a)
            transpose_scale(rows_a, tile_a)
            unit_writes(tile_a, ua, wsem_a, fire=True)

            drain_gathers(idx_b, rows_b, gsem_b)
            transpose_scale(rows_b, tile_b)

            @pl.when(j < n_pairs - 1)
            def _():
                pltpu.sync_copy(idx_hbm.at[pl.ds((ua + 2) * UNIT, UNIT)], idx_a)
                fire_gathers(idx_a, rows_a, gsem_a)

            unit_writes(tile_b, ub, wsem_b, fire=True)
            return carry

        lax.fori_loop(0, n_pairs, pair_body, 0)

        unit_writes(tile_a, unit0 + 2 * n_pairs - 2, wsem_a, fire=False)
        unit_writes(tile_b, unit0 + 2 * n_pairs - 1, wsem_b, fire=False)

    return k(idx_flat, weight)


def kernel(input_ids, weight):
    b, s = input_ids.shape
    idx_flat = (input_ids.astype(jnp.int32)
                .reshape(b // 128, 128, s // 8, 8)
                .transpose(2, 0, 3, 1)
                .reshape(b * s))
    y = _embed_native(idx_flat, weight * jnp.float32(SCALE), b=b, s=s)
    return (y.reshape(s, EMB_DIM // 8, b // 128, 8, 128)
            .transpose(2, 4, 0, 1, 3)
            .reshape(b, s, EMB_DIM))


# speedup vs baseline: 1.4480x; 1.4480x over previous
import functools

import jax
import jax.numpy as jnp
from jax import lax
from jax.experimental import pallas as pl
from jax.experimental.pallas import tpu as pltpu
from jax.experimental.pallas import tpu_sc as plsc

EMB_DIM = 32
SCALE = float(EMB_DIM ** 0.5)

NUM_CORES = 2
NUM_SUBCORES = 16
NUM_WORKERS = NUM_CORES * NUM_SUBCORES
LANES = 16

IDX_PER_STREAM = 128
STREAMS_PER_UNIT = 4
UNIT = IDX_PER_STREAM * STREAMS_PER_UNIT
S8_PER_UNIT = 4
TILE_WORDS = 8 * 128
UNIT_TILES = S8_PER_UNIT * (EMB_DIM // 8)


REF_BLOCKS = 7872


def _reformat_table(wt):
    mesh = plsc.VectorSubcoreMesh(core_axis_name="c", subcore_axis_name="s")
    d32 = wt.shape[0]
    n_per_w = REF_BLOCKS // NUM_WORKERS
    n_pairs = n_per_w // 2
    q_max = (wt.shape[1] + 127) // 128 - 1

    @functools.partial(
        pl.kernel,
        mesh=mesh,
        compiler_params=pltpu.CompilerParams(needs_layout_passes=False),
        out_type=jax.ShapeDtypeStruct((REF_BLOCKS * 128 * d32,), jnp.float32),
        scratch_types=[
            pltpu.VMEM((d32, 129), jnp.float32),
            pltpu.VMEM((d32, 129), jnp.float32),
            pltpu.VMEM((128 * d32,), jnp.float32),
            pltpu.VMEM((128 * d32,), jnp.float32),
            pltpu.SemaphoreType.DMA,
            pltpu.SemaphoreType.DMA,
            pltpu.SemaphoreType.DMA,
            pltpu.SemaphoreType.DMA,
        ],
    )
    def k(wt_hbm, out_hbm, tb_a, tb_b, pb_a, pb_b,
          gsem_a, gsem_b, wsem_a, wsem_b):
        wid = lax.axis_index("s") * NUM_CORES + lax.axis_index("c")
        q0 = wid * n_per_w
        lane = lax.iota(jnp.int32, LANES)
        rvecs = [lane, lane + LANES]

        def tiles_in(q, tb_v, sem, fire):
            qs = jnp.minimum(q, q_max)
            for td in range(d32 // 8):
                src = wt_hbm.at[pl.ds(td * 8, 8), pl.ds(qs * 128, 128)]
                dst = tb_v.at[pl.ds(td * 8, 8), pl.ds(0, 128)]
                if fire:
                    pltpu.async_copy(src, dst, sem)
                else:
                    pltpu.make_async_copy(src, dst, sem).wait()

        def block_out(q, pb_v, sem, fire):
            dst = out_hbm.at[pl.ds(q * (128 * d32), 128 * d32)]
            if fire:
                pltpu.async_copy(pb_v, dst, sem)
            else:
                pltpu.make_async_copy(pb_v, dst, sem).wait()

        def transpose_block(tb_v, pb_v):
            @plsc.parallel_loop(0, 128, 1, unroll=4,
                                carry=jnp.zeros((LANES,), jnp.int32))
            def body(bcol, col):
                for h in range(2):
                    v = plsc.load_gather(tb_v, [rvecs[h], col])
                    pb_v[pl.ds(bcol * d32 + h * LANES, LANES)] = v
                return col + 1

        tiles_in(q0, tb_a, gsem_a, fire=True)

        def pair_body(j, carry):
            qa = q0 + 2 * j
            qb = qa + 1

            @pl.when(j > 0)
            def _():
                block_out(qb - 2, pb_b, wsem_b, fire=False)

            tiles_in(qb, tb_b, gsem_b, fire=True)

            @pl.when(j > 0)
            def _():
                block_out(qa - 2, pb_a, wsem_a, fire=False)

            tiles_in(qa, tb_a, gsem_a, fire=False)
            transpose_block(tb_a, pb_a)
            block_out(qa, pb_a, wsem_a, fire=True)

            tiles_in(qb, tb_b, gsem_b, fire=False)
            transpose_block(tb_b, pb_b)

            @pl.when(j < n_pairs - 1)
            def _():
                tiles_in(qa + 2, tb_a, gsem_a, fire=True)

            block_out(qb, pb_b, wsem_b, fire=True)
            return carry

        lax.fori_loop(0, n_pairs, pair_body, 0)
        block_out(q0 + n_per_w - 2, pb_a, wsem_a, fire=False)
        block_out(q0 + n_per_w - 1, pb_b, wsem_b, fire=False)

    return k(wt)


@functools.partial(jax.jit, static_argnames=("b", "s"))
def _embed_native(idx_flat, weight, *, b, s):
    total = b * s
    n_per_w = total // NUM_WORKERS
    n_units = n_per_w // UNIT
    n_pairs = n_units // 2
    assert n_pairs * 2 * UNIT == n_per_w
    b_tiles = b // 128
    d_tiles = EMB_DIM // 8
    units_per_tile = 8 // S8_PER_UNIT
    out_rows = s * d_tiles * b_tiles
    table = _reformat_table(weight.T).reshape(REF_BLOCKS * 128, EMB_DIM)
    mesh = plsc.VectorSubcoreMesh(core_axis_name="c", subcore_axis_name="s")

    @functools.partial(
        pl.kernel,
        mesh=mesh,
        compiler_params=pltpu.CompilerParams(
            use_tc_tiling_on_sc=False, needs_layout_passes=False),
        out_type=jax.ShapeDtypeStruct((out_rows, 8, 128), jnp.float32),
        scratch_types=[
            pltpu.VMEM((UNIT,), jnp.int32),
            pltpu.VMEM((UNIT,), jnp.int32),
            pltpu.VMEM((UNIT, EMB_DIM), jnp.float32),
            pltpu.VMEM((UNIT, EMB_DIM), jnp.float32),
            pltpu.VMEM((UNIT_TILES * 8, 129), jnp.float32),
            pltpu.VMEM((UNIT_TILES * 8, 129), jnp.float32),
            pltpu.SemaphoreType.DMA,
            pltpu.SemaphoreType.DMA,
            pltpu.SemaphoreType.DMA,
            pltpu.SemaphoreType.DMA,
        ],
    )
    def k(idx_hbm, table_hbm, out_hbm, idx_a, idx_b, rows_a, rows_b,
          tile_a, tile_b, gsem_a, gsem_b, wsem_a, wsem_b):
        wid = lax.axis_index("s") * NUM_CORES + lax.axis_index("c")
        unit0 = wid * n_units

        def fire_gathers(idx_v, rows_v, sem):
            for t in range(STREAMS_PER_UNIT):
                sl = pl.ds(t * IDX_PER_STREAM, IDX_PER_STREAM)
                pltpu.async_copy(table_hbm.at[idx_v.at[sl]], rows_v.at[sl], sem)

        def drain_gathers(idx_v, rows_v, sem):
            for t in range(STREAMS_PER_UNIT):
                sl = pl.ds(t * IDX_PER_STREAM, IDX_PER_STREAM)
                pltpu.make_async_copy(
                    table_hbm.at[idx_v.at[sl]], rows_v.at[sl], sem).wait()

        lane = lax.iota(jnp.int32, LANES)

        def transpose_scale(rows_v, tile_v):
            @plsc.parallel_loop(0, IDX_PER_STREAM, 1, unroll=4,
                                carry=jnp.zeros((LANES,), jnp.int32))
            def body(t128, col):
                for s8 in range(S8_PER_UNIT):
                    tok = s8 * IDX_PER_STREAM + t128
                    for h in range(2):
                        v = rows_v[tok, pl.ds(h * LANES, LANES)] * SCALE
                        plsc.store_scatter(
                            tile_v, [lane + (s8 * 32 + h * LANES), col], v)
                return col + 1

        def unit_coords(u):
            ts = u // (b_tiles * units_per_tile)
            r = lax.rem(u, b_tiles * units_per_tile)
            tb = r // units_per_tile
            h = lax.rem(r, units_per_tile)
            return ts, tb, h

        def unit_writes(tile_v, u, sem, fire):
            ts, tb, h = unit_coords(u)
            for s8 in range(S8_PER_UNIT):
                s_row = ts * 8 + h * S8_PER_UNIT + s8
                for td in range(d_tiles):
                    src = tile_v.at[pl.ds((s8 * d_tiles + td) * 8, 8),
                                    pl.ds(0, 128)]
                    dst = out_hbm.at[(s_row * d_tiles + td) * b_tiles + tb]
                    if fire:
                        pltpu.async_copy(src, dst, sem)
                    else:
                        pltpu.make_async_copy(src, dst, sem).wait()

        pltpu.sync_copy(idx_hbm.at[pl.ds(unit0 * UNIT, UNIT)], idx_a)
        fire_gathers(idx_a, rows_a, gsem_a)

        def pair_body(j, carry):
            ua = unit0 + 2 * j
            ub = ua + 1

            @pl.when(j > 0)
            def _():
                unit_writes(tile_b, ub - 2, wsem_b, fire=False)

            pltpu.sync_copy(idx_hbm.at[pl.ds(ub * UNIT, UNIT)], idx_b)
            fire_gathers(idx_b, rows_b, gsem_b)

            @pl.when(j > 0)
            def _():
                unit_writes(tile_a, ua - 2, wsem_a, fire=False)

            drain_gathers(idx_a, rows_a, gsem_a)
            transpose_scale(rows_a, tile_a)
            unit_writes(tile_a, ua, wsem_a, fire=True)

            drain_gathers(idx_b, rows_b, gsem_b)
            transpose_scale(rows_b, tile_b)

            @pl.when(j < n_pairs - 1)
            def _():
                pltpu.sync_copy(idx_hbm.at[pl.ds((ua + 2) * UNIT, UNIT)], idx_a)
                fire_gathers(idx_a, rows_a, gsem_a)

            unit_writes(tile_b, ub, wsem_b, fire=True)
            return carry

        lax.fori_loop(0, n_pairs, pair_body, 0)

        unit_writes(tile_a, unit0 + 2 * n_pairs - 2, wsem_a, fire=False)
        unit_writes(tile_b, unit0 + 2 * n_pairs - 1, wsem_b, fire=False)

    return k(idx_flat, table)


def kernel(input_ids, weight):
    b, s = input_ids.shape
    idx_flat = (input_ids.astype(jnp.int32)
                .reshape(b // 128, 128, s // 8, 8)
                .transpose(2, 0, 3, 1)
                .reshape(b * s))
    y = _embed_native(idx_flat, weight, b=b, s=s)
    return (y.reshape(s, EMB_DIM // 8, b // 128, 8, 128)
            .transpose(2, 4, 0, 1, 3)
            .reshape(b, s, EMB_DIM))


# speedup vs baseline: 1.5775x; 1.0894x over previous
import functools

import jax
import jax.numpy as jnp
from jax import lax
from jax.experimental import pallas as pl
from jax.experimental.pallas import tpu as pltpu
from jax.experimental.pallas import tpu_sc as plsc

EMB_DIM = 32
SCALE = float(EMB_DIM ** 0.5)

NUM_CORES = 2
NUM_SUBCORES = 16
NUM_WORKERS = NUM_CORES * NUM_SUBCORES
LANES = 16

IDX_PER_STREAM = 128
STREAMS_PER_UNIT = 4
UNIT = IDX_PER_STREAM * STREAMS_PER_UNIT
S8_PER_UNIT = 4
TILE_WORDS = 8 * 128
UNIT_TILES = S8_PER_UNIT * (EMB_DIM // 8)


REF_G = 6
REF_NB = 42
REF_ROWS = 7813 * 128


def _reformat_table(wt):
    mesh = plsc.VectorSubcoreMesh(core_axis_name="c", subcore_axis_name="s")
    d32 = wt.shape[0]
    n_pairs = REF_NB // 2
    bw = REF_G * 128
    bwords = bw * d32
    row_max = REF_ROWS - bw

    @functools.partial(
        pl.kernel,
        mesh=mesh,
        compiler_params=pltpu.CompilerParams(needs_layout_passes=False),
        out_type=jax.ShapeDtypeStruct((REF_ROWS * d32,), jnp.float32),
        scratch_types=[
            pltpu.VMEM((d32, bw + 1), jnp.float32),
            pltpu.VMEM((d32, bw + 1), jnp.float32),
            pltpu.VMEM((bwords,), jnp.float32),
            pltpu.VMEM((bwords,), jnp.float32),
            pltpu.SemaphoreType.DMA,
            pltpu.SemaphoreType.DMA,
            pltpu.SemaphoreType.DMA,
            pltpu.SemaphoreType.DMA,
        ],
    )
    def k(wt_hbm, out_hbm, tb_a, tb_b, pb_a, pb_b,
          gsem_a, gsem_b, wsem_a, wsem_b):
        wid = lax.axis_index("s") * NUM_CORES + lax.axis_index("c")
        k0 = wid * REF_NB
        lane = lax.iota(jnp.int32, LANES)
        rvecs = [lane, lane + LANES]

        def row0_of(kk):
            return jnp.minimum(kk * bw, row_max)

        def tiles_in(kk, tb_v, sem, fire):
            r0 = row0_of(kk)
            for td in range(d32 // 8):
                src = wt_hbm.at[pl.ds(td * 8, 8), pl.ds(r0, bw)]
                dst = tb_v.at[pl.ds(td * 8, 8), pl.ds(0, bw)]
                if fire:
                    pltpu.async_copy(src, dst, sem)
                else:
                    pltpu.make_async_copy(src, dst, sem).wait()

        def block_out(kk, pb_v, sem, fire):
            dst = out_hbm.at[pl.ds(row0_of(kk) * d32, bwords)]
            if fire:
                pltpu.async_copy(pb_v, dst, sem)
            else:
                pltpu.make_async_copy(pb_v, dst, sem).wait()

        def transpose_block(tb_v, pb_v):
            @plsc.parallel_loop(0, bw, 1, unroll=4,
                                carry=jnp.zeros((LANES,), jnp.int32))
            def body(bcol, col):
                for h in range(2):
                    v = plsc.load_gather(tb_v, [rvecs[h], col])
                    pb_v[pl.ds(bcol * d32 + h * LANES, LANES)] = v
                return col + 1

        tiles_in(k0, tb_a, gsem_a, fire=True)

        def pair_body(j, carry):
            ka = k0 + 2 * j
            kb = ka + 1

            @pl.when(j > 0)
            def _():
                block_out(kb - 2, pb_b, wsem_b, fire=False)

            tiles_in(kb, tb_b, gsem_b, fire=True)

            @pl.when(j > 0)
            def _():
                block_out(ka - 2, pb_a, wsem_a, fire=False)

            tiles_in(ka, tb_a, gsem_a, fire=False)
            transpose_block(tb_a, pb_a)
            block_out(ka, pb_a, wsem_a, fire=True)

            tiles_in(kb, tb_b, gsem_b, fire=False)
            transpose_block(tb_b, pb_b)

            @pl.when(j < n_pairs - 1)
            def _():
                tiles_in(ka + 2, tb_a, gsem_a, fire=True)

            block_out(kb, pb_b, wsem_b, fire=True)
            return carry

        lax.fori_loop(0, n_pairs, pair_body, 0)
        block_out(k0 + REF_NB - 2, pb_a, wsem_a, fire=False)
        block_out(k0 + REF_NB - 1, pb_b, wsem_b, fire=False)

    return k(wt)


@functools.partial(jax.jit, static_argnames=("b", "s"))
def _embed_native(idx_flat, weight, *, b, s):
    total = b * s
    n_per_w = total // NUM_WORKERS
    n_units = n_per_w // UNIT
    n_pairs = n_units // 2
    assert n_pairs * 2 * UNIT == n_per_w
    b_tiles = b // 128
    d_tiles = EMB_DIM // 8
    units_per_tile = 8 // S8_PER_UNIT
    out_rows = s * d_tiles * b_tiles
    table = _reformat_table(weight.T).reshape(REF_ROWS, EMB_DIM)
    mesh = plsc.VectorSubcoreMesh(core_axis_name="c", subcore_axis_name="s")

    @functools.partial(
        pl.kernel,
        mesh=mesh,
        compiler_params=pltpu.CompilerParams(
            use_tc_tiling_on_sc=False, needs_layout_passes=False),
        out_type=jax.ShapeDtypeStruct((out_rows, 8, 128), jnp.float32),
        scratch_types=[
            pltpu.VMEM((UNIT,), jnp.int32),
            pltpu.VMEM((UNIT,), jnp.int32),
            pltpu.VMEM((UNIT, EMB_DIM), jnp.float32),
            pltpu.VMEM((UNIT, EMB_DIM), jnp.float32),
            pltpu.VMEM((UNIT_TILES * 8, 129), jnp.float32),
            pltpu.VMEM((UNIT_TILES * 8, 129), jnp.float32),
            pltpu.SemaphoreType.DMA,
            pltpu.SemaphoreType.DMA,
            pltpu.SemaphoreType.DMA,
            pltpu.SemaphoreType.DMA,
        ],
    )
    def k(idx_hbm, table_hbm, out_hbm, idx_a, idx_b, rows_a, rows_b,
          tile_a, tile_b, gsem_a, gsem_b, wsem_a, wsem_b):
        wid = lax.axis_index("s") * NUM_CORES + lax.axis_index("c")
        unit0 = wid * n_units

        def fire_gathers(idx_v, rows_v, sem):
            for t in range(STREAMS_PER_UNIT):
                sl = pl.ds(t * IDX_PER_STREAM, IDX_PER_STREAM)
                pltpu.async_copy(table_hbm.at[idx_v.at[sl]], rows_v.at[sl], sem)

        def drain_gathers(idx_v, rows_v, sem):
            for t in range(STREAMS_PER_UNIT):
                sl = pl.ds(t * IDX_PER_STREAM, IDX_PER_STREAM)
                pltpu.make_async_copy(
                    table_hbm.at[idx_v.at[sl]], rows_v.at[sl], sem).wait()

        lane = lax.iota(jnp.int32, LANES)

        def transpose_scale(rows_v, tile_v):
            @plsc.parallel_loop(0, IDX_PER_STREAM, 1, unroll=4,
                                carry=jnp.zeros((LANES,), jnp.int32))
            def body(t128, col):
                for s8 in range(S8_PER_UNIT):
                    tok = s8 * IDX_PER_STREAM + t128
                    for h in range(2):
                        v = rows_v[tok, pl.ds(h * LANES, LANES)] * SCALE
                        plsc.store_scatter(
                            tile_v, [lane + (s8 * 32 + h * LANES), col], v)
                return col + 1

        def unit_coords(u):
            ts = u // (b_tiles * units_per_tile)
            r = lax.rem(u, b_tiles * units_per_tile)
            tb = r // units_per_tile
            h = lax.rem(r, units_per_tile)
            return ts, tb, h

        def unit_writes(tile_v, u, sem, fire):
            ts, tb, h = unit_coords(u)
            for s8 in range(S8_PER_UNIT):
                s_row = ts * 8 + h * S8_PER_UNIT + s8
                for td in range(d_tiles):
                    src = tile_v.at[pl.ds((s8 * d_tiles + td) * 8, 8),
                                    pl.ds(0, 128)]
                    dst = out_hbm.at[(s_row * d_tiles + td) * b_tiles + tb]
                    if fire:
                        pltpu.async_copy(src, dst, sem)
                    else:
                        pltpu.make_async_copy(src, dst, sem).wait()

        pltpu.sync_copy(idx_hbm.at[pl.ds(unit0 * UNIT, UNIT)], idx_a)
        fire_gathers(idx_a, rows_a, gsem_a)

        def pair_body(j, carry):
            ua = unit0 + 2 * j
            ub = ua + 1

            @pl.when(j > 0)
            def _():
                unit_writes(tile_b, ub - 2, wsem_b, fire=False)

            pltpu.sync_copy(idx_hbm.at[pl.ds(ub * UNIT, UNIT)], idx_b)
            fire_gathers(idx_b, rows_b, gsem_b)

            @pl.when(j > 0)
            def _():
                unit_writes(tile_a, ua - 2, wsem_a, fire=False)

            drain_gathers(idx_a, rows_a, gsem_a)
            transpose_scale(rows_a, tile_a)
            unit_writes(tile_a, ua, wsem_a, fire=True)

            drain_gathers(idx_b, rows_b, gsem_b)
            transpose_scale(rows_b, tile_b)

            @pl.when(j < n_pairs - 1)
            def _():
                pltpu.sync_copy(idx_hbm.at[pl.ds((ua + 2) * UNIT, UNIT)], idx_a)
                fire_gathers(idx_a, rows_a, gsem_a)

            unit_writes(tile_b, ub, wsem_b, fire=True)
            return carry

        lax.fori_loop(0, n_pairs, pair_body, 0)

        unit_writes(tile_a, unit0 + 2 * n_pairs - 2, wsem_a, fire=False)
        unit_writes(tile_b, unit0 + 2 * n_pairs - 1, wsem_b, fire=False)

    return k(idx_flat, table)


def kernel(input_ids, weight):
    b, s = input_ids.shape
    idx_flat = (input_ids.astype(jnp.int32)
                .reshape(b // 128, 128, s // 8, 8)
                .transpose(2, 0, 3, 1)
                .reshape(b * s))
    y = _embed_native(idx_flat, weight, b=b, s=s)
    return (y.reshape(s, EMB_DIM // 8, b // 128, 8, 128)
            .transpose(2, 4, 0, 1, 3)
            .reshape(b, s, EMB_DIM))


# speedup vs baseline: 1.5777x; 1.0002x over previous
import functools

import jax
import jax.numpy as jnp
from jax import lax
from jax.experimental import pallas as pl
from jax.experimental.pallas import tpu as pltpu
from jax.experimental.pallas import tpu_sc as plsc

EMB_DIM = 32
SCALE = float(EMB_DIM ** 0.5)

NUM_CORES = 2
NUM_SUBCORES = 16
NUM_WORKERS = NUM_CORES * NUM_SUBCORES
LANES = 16

IDX_PER_STREAM = 128
STREAMS_PER_UNIT = 4
UNIT = IDX_PER_STREAM * STREAMS_PER_UNIT
S8_PER_UNIT = 4
TILE_WORDS = 8 * 128
UNIT_TILES = S8_PER_UNIT * (EMB_DIM // 8)


REF_G = 7
REF_NB = 36
REF_ROWS = 7813 * 128


def _reformat_table(wt):
    mesh = plsc.VectorSubcoreMesh(core_axis_name="c", subcore_axis_name="s")
    d32 = wt.shape[0]
    n_pairs = REF_NB // 2
    bw = REF_G * 128
    bwords = bw * d32
    row_max = REF_ROWS - bw

    @functools.partial(
        pl.kernel,
        mesh=mesh,
        compiler_params=pltpu.CompilerParams(needs_layout_passes=False),
        out_type=jax.ShapeDtypeStruct((REF_ROWS * d32,), jnp.float32),
        scratch_types=[
            pltpu.VMEM((d32, bw + 1), jnp.float32),
            pltpu.VMEM((d32, bw + 1), jnp.float32),
            pltpu.VMEM((bwords,), jnp.float32),
            pltpu.VMEM((bwords,), jnp.float32),
            pltpu.SemaphoreType.DMA,
            pltpu.SemaphoreType.DMA,
            pltpu.SemaphoreType.DMA,
            pltpu.SemaphoreType.DMA,
        ],
    )
    def k(wt_hbm, out_hbm, tb_a, tb_b, pb_a, pb_b,
          gsem_a, gsem_b, wsem_a, wsem_b):
        wid = lax.axis_index("s") * NUM_CORES + lax.axis_index("c")
        k0 = wid * REF_NB
        lane = lax.iota(jnp.int32, LANES)
        rvecs = [lane, lane + LANES]

        def row0_of(kk):
            return jnp.minimum(kk * bw, row_max)

        def tiles_in(kk, tb_v, sem, fire):
            r0 = row0_of(kk)
            for td in range(d32 // 8):
                src = wt_hbm.at[pl.ds(td * 8, 8), pl.ds(r0, bw)]
                dst = tb_v.at[pl.ds(td * 8, 8), pl.ds(0, bw)]
                if fire:
                    pltpu.async_copy(src, dst, sem)
                else:
                    pltpu.make_async_copy(src, dst, sem).wait()

        def block_out(kk, pb_v, sem, fire):
            dst = out_hbm.at[pl.ds(row0_of(kk) * d32, bwords)]
            if fire:
                pltpu.async_copy(pb_v, dst, sem)
            else:
                pltpu.make_async_copy(pb_v, dst, sem).wait()

        def transpose_block(tb_v, pb_v):
            @plsc.parallel_loop(0, bw, 1, unroll=4,
                                carry=jnp.zeros((LANES,), jnp.int32))
            def body(bcol, col):
                for h in range(2):
                    v = plsc.load_gather(tb_v, [rvecs[h], col])
                    pb_v[pl.ds(bcol * d32 + h * LANES, LANES)] = v
                return col + 1

        tiles_in(k0, tb_a, gsem_a, fire=True)

        def pair_body(j, carry):
            ka = k0 + 2 * j
            kb = ka + 1

            @pl.when(j > 0)
            def _():
                block_out(kb - 2, pb_b, wsem_b, fire=False)

            tiles_in(kb, tb_b, gsem_b, fire=True)

            @pl.when(j > 0)
            def _():
                block_out(ka - 2, pb_a, wsem_a, fire=False)

            tiles_in(ka, tb_a, gsem_a, fire=False)
            transpose_block(tb_a, pb_a)
            block_out(ka, pb_a, wsem_a, fire=True)

            tiles_in(kb, tb_b, gsem_b, fire=False)
            transpose_block(tb_b, pb_b)

            @pl.when(j < n_pairs - 1)
            def _():
                tiles_in(ka + 2, tb_a, gsem_a, fire=True)

            block_out(kb, pb_b, wsem_b, fire=True)
            return carry

        lax.fori_loop(0, n_pairs, pair_body, 0)
        block_out(k0 + REF_NB - 2, pb_a, wsem_a, fire=False)
        block_out(k0 + REF_NB - 1, pb_b, wsem_b, fire=False)

    return k(wt)


@functools.partial(jax.jit, static_argnames=("b", "s"))
def _embed_native(idx_flat, weight, *, b, s):
    total = b * s
    n_per_w = total // NUM_WORKERS
    n_units = n_per_w // UNIT
    n_pairs = n_units // 2
    assert n_pairs * 2 * UNIT == n_per_w
    b_tiles = b // 128
    d_tiles = EMB_DIM // 8
    units_per_tile = 8 // S8_PER_UNIT
    out_rows = s * d_tiles * b_tiles
    table = _reformat_table(weight.T).reshape(REF_ROWS, EMB_DIM)
    mesh = plsc.VectorSubcoreMesh(core_axis_name="c", subcore_axis_name="s")

    @functools.partial(
        pl.kernel,
        mesh=mesh,
        compiler_params=pltpu.CompilerParams(
            use_tc_tiling_on_sc=False, needs_layout_passes=False),
        out_type=jax.ShapeDtypeStruct((out_rows, 8, 128), jnp.float32),
        scratch_types=[
            pltpu.VMEM((UNIT,), jnp.int32),
            pltpu.VMEM((UNIT,), jnp.int32),
            pltpu.VMEM((UNIT, EMB_DIM), jnp.float32),
            pltpu.VMEM((UNIT, EMB_DIM), jnp.float32),
            pltpu.VMEM((UNIT_TILES * 8, 129), jnp.float32),
            pltpu.VMEM((UNIT_TILES * 8, 129), jnp.float32),
            pltpu.SemaphoreType.DMA,
            pltpu.SemaphoreType.DMA,
            pltpu.SemaphoreType.DMA,
            pltpu.SemaphoreType.DMA,
        ],
    )
    def k(idx_hbm, table_hbm, out_hbm, idx_a, idx_b, rows_a, rows_b,
          tile_a, tile_b, gsem_a, gsem_b, wsem_a, wsem_b):
        wid = lax.axis_index("s") * NUM_CORES + lax.axis_index("c")
        unit0 = wid * n_units

        def fire_gathers(idx_v, rows_v, sem):
            for t in range(STREAMS_PER_UNIT):
                sl = pl.ds(t * IDX_PER_STREAM, IDX_PER_STREAM)
                pltpu.async_copy(table_hbm.at[idx_v.at[sl]], rows_v.at[sl], sem)

        def drain_gathers(idx_v, rows_v, sem):
            for t in range(STREAMS_PER_UNIT):
                sl = pl.ds(t * IDX_PER_STREAM, IDX_PER_STREAM)
                pltpu.make_async_copy(
                    table_hbm.at[idx_v.at[sl]], rows_v.at[sl], sem).wait()

        lane = lax.iota(jnp.int32, LANES)

        def transpose_scale(rows_v, tile_v):
            @plsc.parallel_loop(0, IDX_PER_STREAM, 1, unroll=4,
                                carry=jnp.zeros((LANES,), jnp.int32))
            def body(t128, col):
                for s8 in range(S8_PER_UNIT):
                    tok = s8 * IDX_PER_STREAM + t128
                    for h in range(2):
                        v = rows_v[tok, pl.ds(h * LANES, LANES)] * SCALE
                        plsc.store_scatter(
                            tile_v, [lane + (s8 * 32 + h * LANES), col], v)
                return col + 1

        def unit_coords(u):
            ts = u // (b_tiles * units_per_tile)
            r = lax.rem(u, b_tiles * units_per_tile)
            tb = r // units_per_tile
            h = lax.rem(r, units_per_tile)
            return ts, tb, h

        def unit_writes(tile_v, u, sem, fire):
            ts, tb, h = unit_coords(u)
            for s8 in range(S8_PER_UNIT):
                s_row = ts * 8 + h * S8_PER_UNIT + s8
                for td in range(d_tiles):
                    src = tile_v.at[pl.ds((s8 * d_tiles + td) * 8, 8),
                                    pl.ds(0, 128)]
                    dst = out_hbm.at[(s_row * d_tiles + td) * b_tiles + tb]
                    if fire:
                        pltpu.async_copy(src, dst, sem)
                    else:
                        pltpu.make_async_copy(src, dst, sem).wait()

        pltpu.sync_copy(idx_hbm.at[pl.ds(unit0 * UNIT, UNIT)], idx_a)
        fire_gathers(idx_a, rows_a, gsem_a)

        def pair_body(j, carry):
            ua = unit0 + 2 * j
            ub = ua + 1

            @pl.when(j > 0)
            def _():
                unit_writes(tile_b, ub - 2, wsem_b, fire=False)

            pltpu.sync_copy(idx_hbm.at[pl.ds(ub * UNIT, UNIT)], idx_b)
            fire_gathers(idx_b, rows_b, gsem_b)

            @pl.when(j > 0)
            def _():
                unit_writes(tile_a, ua - 2, wsem_a, fire=False)

            drain_gathers(idx_a, rows_a, gsem_a)
            transpose_scale(rows_a, tile_a)
            unit_writes(tile_a, ua, wsem_a, fire=True)

            drain_gathers(idx_b, rows_b, gsem_b)
            transpose_scale(rows_b, tile_b)

            @pl.when(j < n_pairs - 1)
            def _():
                pltpu.sync_copy(idx_hbm.at[pl.ds((ua + 2) * UNIT, UNIT)], idx_a)
                fire_gathers(idx_a, rows_a, gsem_a)

            unit_writes(tile_b, ub, wsem_b, fire=True)
            return carry

        lax.fori_loop(0, n_pairs, pair_body, 0)

        unit_writes(tile_a, unit0 + 2 * n_pairs - 2, wsem_a, fire=False)
        unit_writes(tile_b, unit0 + 2 * n_pairs - 1, wsem_b, fire=False)

    return k(idx_flat, table)


def kernel(input_ids, weight):
    b, s = input_ids.shape
    idx_flat = (input_ids.astype(jnp.int32)
                .reshape(b // 128, 128, s // 8, 8)
                .transpose(2, 0, 3, 1)
                .reshape(b * s))
    y = _embed_native(idx_flat, weight, b=b, s=s)
    return (y.reshape(s, EMB_DIM // 8, b // 128, 8, 128)
            .transpose(2, 4, 0, 1, 3)
            .reshape(b, s, EMB_DIM))
